# baseline (speedup 1.0000x reference)
import functools

import jax
import jax.numpy as jnp
from jax import lax
from jax.experimental import pallas as pl
from jax.experimental.pallas import tpu as pltpu
from jax.experimental.pallas import tpu_sc as plsc

ALPHA = 0.75
LN2 = 0.6931471805599453
_LOG2_POLY = (
    -2.14494063, 3.02947821, -1.03925816, 0.15544586,
)

NUM_CORES = 2
NUM_SUBCORES = 16
NW = NUM_CORES * NUM_SUBCORES
LANES = 16
BLOCK = 128
CHUNK = 6144
UNROLL = 1


def _focal16(z, is1):
    e = jnp.exp(-z)
    p = 1.0 / (1.0 + e)
    p = jnp.maximum(p, 1e-4)
    bits = plsc.bitcast(p, jnp.int32)
    ex = lax.shift_right_logical(bits, 23) - 127
    man = plsc.bitcast(
        lax.bitwise_or(lax.bitwise_and(bits, 0x007FFFFF), 0x3F800000),
        jnp.float32)
    poly = jnp.full((LANES,), _LOG2_POLY[-1], dtype=jnp.float32)
    for c in _LOG2_POLY[-2::-1]:
        poly = poly * man + c
    lnp2 = ex.astype(jnp.float32) + poly
    om = 1.0 - p
    na = jnp.where(is1, -ALPHA * LN2, -(1.0 - ALPHA) * LN2)
    return na * om * om * lnp2


def _sc_body(x_hbm, t_hbm, o_hbm, xv, tv, ov, sin, sout):
    wid = lax.axis_index("s") * NUM_CORES + lax.axis_index("c")
    rows_per_w = o_hbm.shape[0] // NW
    nchunks = rows_per_w // CHUNK
    base = wid * rows_per_w

    def start_in(b, ci):
        r0 = base + ci * CHUNK
        pltpu.async_copy(x_hbm.at[pl.ds(2 * r0, 2 * CHUNK)], xv[b], sin[b])
        pltpu.async_copy(t_hbm.at[pl.ds(r0, CHUNK)], tv[b], sin[b])

    def compute(b):
        xb, tb, ob = xv[b], tv[b], ov[b]

        @plsc.parallel_loop(0, CHUNK // BLOCK, 1, unroll=UNROLL)
        def _inner(blk):
            for k in range(BLOCK // LANES):
                x0 = xb[pl.ds(2 * BLOCK * blk + LANES * k, LANES)]
                x1 = xb[pl.ds(2 * BLOCK * blk + BLOCK + LANES * k, LANES)]
                t = tb[pl.ds(BLOCK * blk + LANES * k, LANES)]
                is1 = t == 1
                d = x1 - x0
                z = jnp.where(is1, d, -d)
                ob[pl.ds(BLOCK * blk + LANES * k, LANES)] = _focal16(z, is1)

    def start_out(b, ci):
        r0 = base + ci * CHUNK
        pltpu.async_copy(ov[b], o_hbm.at[pl.ds(r0, CHUNK)], sout[b])

    def wait_in(b, ci):
        r0 = base + ci * CHUNK
        pltpu.make_async_copy(x_hbm.at[pl.ds(2 * r0, 2 * CHUNK)], xv[b],
                              sin[b]).wait()
        pltpu.make_async_copy(t_hbm.at[pl.ds(r0, CHUNK)], tv[b],
                              sin[b]).wait()

    def wait_out(b, ci):
        r0 = base + ci * CHUNK
        pltpu.make_async_copy(ov[b], o_hbm.at[pl.ds(r0, CHUNK)],
                              sout[b]).wait()

    start_in(0, 0)
    start_in(1, 1)
    wait_in(0, 0)
    compute(0)
    start_out(0, 0)
    wait_in(1, 1)
    compute(1)
    start_out(1, 1)
    wait_out(0, 0)
    wait_out(1, 1)


TC_BLOCK = 256


def _tc_body(x_ref, t_ref, o_ref):
    x = x_ref[...].reshape(TC_BLOCK, 2, BLOCK)
    x0 = x[:, 0, :]
    x1 = x[:, 1, :]
    t = t_ref[...]
    is1 = t == 1
    d = x1 - x0
    z = jnp.where(is1, d, -d)
    p = 1.0 / (1.0 + jnp.exp(-z))
    p = jnp.minimum(jnp.maximum(p, 1e-4), 1.0)
    om = 1.0 - p
    na = jnp.where(is1, -ALPHA, -(1.0 - ALPHA))
    o_ref[...] = na * om * om * jnp.log(p)


def kernel(input, target):
    n = input.shape[0]
    flat = input.reshape(n // BLOCK, BLOCK, 2).swapaxes(1, 2).reshape(-1)
    tgt = target.astype(jnp.int32)

    s = 2 * NW * CHUNK

    mesh = plsc.VectorSubcoreMesh(core_axis_name="c", subcore_axis_name="s")
    run = functools.partial(
        pl.kernel,
        out_type=jax.ShapeDtypeStruct((s,), jnp.float32),
        mesh=mesh,
        scratch_types=[
            [pltpu.VMEM((2 * CHUNK,), jnp.float32) for _ in range(2)],
            [pltpu.VMEM((CHUNK,), jnp.int32) for _ in range(2)],
            [pltpu.VMEM((CHUNK,), jnp.float32) for _ in range(2)],
            [pltpu.SemaphoreType.DMA for _ in range(2)],
            [pltpu.SemaphoreType.DMA for _ in range(2)],
        ],
        compiler_params=pltpu.CompilerParams(needs_layout_passes=False),
    )(_sc_body)
    sc_out = run(flat, tgt)

    r2 = (n - s) // BLOCK
    off = s // (BLOCK * TC_BLOCK)
    tc_out = pl.pallas_call(
        _tc_body,
        grid=(r2 // TC_BLOCK,),
        in_specs=[
            pl.BlockSpec((2 * TC_BLOCK, BLOCK), lambda i: (i + off, 0)),
            pl.BlockSpec((TC_BLOCK, BLOCK), lambda i: (i + off, 0)),
        ],
        out_specs=pl.BlockSpec((TC_BLOCK, BLOCK), lambda i: (i, 0)),
        out_shape=jax.ShapeDtypeStruct((r2, BLOCK), jnp.float32),
    )(flat.reshape(2 * n // BLOCK, BLOCK), tgt.reshape(n // BLOCK, BLOCK))

    out = jnp.concatenate([sc_out, tc_out.reshape(n - s)])
    return out.reshape(n, 1)


# speedup vs baseline: 1.0408x; 1.0408x over previous
import functools

import jax
import jax.numpy as jnp
from jax import lax
from jax.experimental import pallas as pl
from jax.experimental.pallas import tpu as pltpu
from jax.experimental.pallas import tpu_sc as plsc

ALPHA = 0.75
LN2 = 0.6931471805599453
_LOG2_POLY = (
    -2.14494063, 3.02947821, -1.03925816, 0.15544586,
)

NUM_CORES = 2
NUM_SUBCORES = 16
NW = NUM_CORES * NUM_SUBCORES
LANES = 16
BLOCK = 128
CHUNK = 7168
UNROLL = 1


def _focal16(z, is1):
    e = jnp.exp(-z)
    p = 1.0 / (1.0 + e)
    p = jnp.maximum(p, 1e-4)
    bits = plsc.bitcast(p, jnp.int32)
    ex = lax.shift_right_logical(bits, 23) - 127
    man = plsc.bitcast(
        lax.bitwise_or(lax.bitwise_and(bits, 0x007FFFFF), 0x3F800000),
        jnp.float32)
    poly = jnp.full((LANES,), _LOG2_POLY[-1], dtype=jnp.float32)
    for c in _LOG2_POLY[-2::-1]:
        poly = poly * man + c
    lnp2 = ex.astype(jnp.float32) + poly
    om = 1.0 - p
    na = jnp.where(is1, -ALPHA * LN2, -(1.0 - ALPHA) * LN2)
    return na * om * om * lnp2


def _sc_body(x_hbm, t_hbm, o_hbm, xv, tv, ov, sin, sout):
    wid = lax.axis_index("s") * NUM_CORES + lax.axis_index("c")
    rows_per_w = o_hbm.shape[0] // NW
    nchunks = rows_per_w // CHUNK
    base = wid * rows_per_w

    def start_in(b, ci):
        r0 = base + ci * CHUNK
        pltpu.async_copy(x_hbm.at[pl.ds(2 * r0, 2 * CHUNK)], xv[b], sin[b])
        pltpu.async_copy(t_hbm.at[pl.ds(r0, CHUNK)], tv[b], sin[b])

    def compute(b):
        xb, tb, ob = xv[b], tv[b], ov[b]

        @plsc.parallel_loop(0, CHUNK // BLOCK, 1, unroll=UNROLL)
        def _inner(blk):
            for k in range(BLOCK // LANES):
                x0 = xb[pl.ds(2 * BLOCK * blk + LANES * k, LANES)]
                x1 = xb[pl.ds(2 * BLOCK * blk + BLOCK + LANES * k, LANES)]
                t = tb[pl.ds(BLOCK * blk + LANES * k, LANES)]
                is1 = t == 1
                d = x1 - x0
                z = jnp.where(is1, d, -d)
                ob[pl.ds(BLOCK * blk + LANES * k, LANES)] = _focal16(z, is1)

    def start_out(b, ci):
        r0 = base + ci * CHUNK
        pltpu.async_copy(ov[b], o_hbm.at[pl.ds(r0, CHUNK)], sout[b])

    def wait_in(b, ci):
        r0 = base + ci * CHUNK
        pltpu.make_async_copy(x_hbm.at[pl.ds(2 * r0, 2 * CHUNK)], xv[b],
                              sin[b]).wait()
        pltpu.make_async_copy(t_hbm.at[pl.ds(r0, CHUNK)], tv[b],
                              sin[b]).wait()

    def wait_out(b, ci):
        r0 = base + ci * CHUNK
        pltpu.make_async_copy(ov[b], o_hbm.at[pl.ds(r0, CHUNK)],
                              sout[b]).wait()

    start_in(0, 0)
    start_in(1, 1)
    wait_in(0, 0)
    compute(0)
    start_out(0, 0)
    wait_in(1, 1)
    compute(1)
    start_out(1, 1)
    wait_out(0, 0)
    wait_out(1, 1)


TC_BLOCK = 256


def _tc_body(x_ref, t_ref, o_ref):
    x = x_ref[...].reshape(TC_BLOCK, 2, BLOCK)
    x0 = x[:, 0, :]
    x1 = x[:, 1, :]
    t = t_ref[...]
    is1 = t == 1
    d = x1 - x0
    z = jnp.where(is1, d, -d)
    p = 1.0 / (1.0 + jnp.exp(-z))
    p = jnp.minimum(jnp.maximum(p, 1e-4), 1.0)
    om = 1.0 - p
    na = jnp.where(is1, -ALPHA, -(1.0 - ALPHA))
    o_ref[...] = na * om * om * jnp.log(p)


def kernel(input, target):
    n = input.shape[0]
    flat = input.reshape(n // BLOCK, BLOCK, 2).swapaxes(1, 2).reshape(-1)
    tgt = target.astype(jnp.int32)

    s = 2 * NW * CHUNK

    mesh = plsc.VectorSubcoreMesh(core_axis_name="c", subcore_axis_name="s")
    run = functools.partial(
        pl.kernel,
        out_type=jax.ShapeDtypeStruct((s,), jnp.float32),
        mesh=mesh,
        scratch_types=[
            [pltpu.VMEM((2 * CHUNK,), jnp.float32) for _ in range(2)],
            [pltpu.VMEM((CHUNK,), jnp.int32) for _ in range(2)],
            [pltpu.VMEM((CHUNK,), jnp.float32) for _ in range(2)],
            [pltpu.SemaphoreType.DMA for _ in range(2)],
            [pltpu.SemaphoreType.DMA for _ in range(2)],
        ],
        compiler_params=pltpu.CompilerParams(needs_layout_passes=False),
    )(_sc_body)
    sc_out = run(flat, tgt)

    r2 = (n - s) // BLOCK
    off = s // (BLOCK * TC_BLOCK)
    tc_out = pl.pallas_call(
        _tc_body,
        grid=(r2 // TC_BLOCK,),
        in_specs=[
            pl.BlockSpec((2 * TC_BLOCK, BLOCK), lambda i: (i + off, 0)),
            pl.BlockSpec((TC_BLOCK, BLOCK), lambda i: (i + off, 0)),
        ],
        out_specs=pl.BlockSpec((TC_BLOCK, BLOCK), lambda i: (i, 0)),
        out_shape=jax.ShapeDtypeStruct((r2, BLOCK), jnp.float32),
    )(flat.reshape(2 * n // BLOCK, BLOCK), tgt.reshape(n // BLOCK, BLOCK))

    out = jnp.concatenate([sc_out, tc_out.reshape(n - s)])
    return out.reshape(n, 1)
